# baseline (speedup 1.0000x reference)
import jax
import jax.numpy as jnp
from jax import lax
from jax.experimental import pallas as pl
from jax.experimental.pallas import tpu as pltpu
from jax.experimental.pallas import tpu_sc as plsc

F = 32
B_USER = 16384
B_ITEM = 4096
NV = 1000000
NC = 2
NS = 16
L = 16
NW = NC * NS
IPW = B_ITEM // NW
UPW = B_USER // NW
CH = 128
N_UCH = UPW // CH
NBUF = 16

_SC_PARAMS = pltpu.CompilerParams(
    needs_layout_passes=False, use_tc_tiling_on_sc=True)
_SC_MESH = plsc.VectorSubcoreMesh(core_axis_name="c", subcore_axis_name="s")


MAXC = (NV - CH) // CH * CH
TAIL = NV // CH * CH


def _item_body(itT, iidx, s2_out, idx_sm, part_v, ps_v, shared, sem, tsem,
               tail_v, *blks):
  cid = lax.axis_index("c")
  sid = lax.axis_index("s")
  wid = sid * NC + cid
  zero = jnp.zeros((L,), jnp.float32)
  lane = lax.iota(jnp.int32, L)

  pltpu.sync_copy(iidx.at[pl.ds(wid * IPW, IPW)], idx_sm)
  pltpu.async_copy(itT.at[:, pl.ds(TAIL, NV - TAIL)], tail_v, tsem).wait()

  js = []
  for b in range(IPW // L):
    jv = idx_sm[pl.ds(b * L, L)]
    js.extend(jv[l] for l in range(L))

  def aligned_col(j):
    return pl.multiple_of(
        jnp.minimum(j & ~jnp.int32(CH - 1), jnp.int32(MAXC)), CH)

  def fire(k):
    col = aligned_col(js[k])
    return pltpu.async_copy(itT.at[:, pl.ds(col, CH)], blks[k % NBUF], sem)

  copies = [fire(k) for k in range(NBUF)]
  a0, a1 = zero, zero
  for k in range(IPW):
    copies[k % NBUF].wait()
    j = js[k]
    col = aligned_col(j)
    is_tail = j >= TAIL
    sub = jnp.full((L,), jnp.minimum(j - col, CH - 1), jnp.int32)
    tsub = jnp.full((L,), jnp.clip(j - TAIL, 0, NV - TAIL - 1), jnp.int32)
    m0 = plsc.load_gather(blks[k % NBUF], [lane, sub])
    m1 = plsc.load_gather(blks[k % NBUF], [lane + L, sub])
    t0 = plsc.load_gather(tail_v, [lane, tsub])
    t1 = plsc.load_gather(tail_v, [lane + L, tsub])
    a0 = a0 + jnp.where(is_tail, t0, m0)
    a1 = a1 + jnp.where(is_tail, t1, m1)
    if k + NBUF < IPW:
      copies[k % NBUF] = fire(k + NBUF)

  part_v[pl.ds(0, L)] = a0
  part_v[pl.ds(L, L)] = a1
  pltpu.sync_copy(part_v, shared.at[pl.ds(sid * F, F)])
  plsc.subcore_barrier()
  pltpu.sync_copy(shared, ps_v)

  @pl.loop(0, NS, init_carry=(zero, zero), unroll=True)
  def _part_acc(i, carry):
    b0, b1 = carry
    return (b0 + ps_v[pl.ds(i * F, L)], b1 + ps_v[pl.ds(i * F + L, L)])
  s0, s1 = _part_acc

  @pl.when(sid == 0)
  def _():
    part_v[pl.ds(0, L)] = s0
    part_v[pl.ds(L, L)] = s1
    pltpu.sync_copy(part_v, s2_out.at[pl.ds(cid * F, F)])


_item_kernel = pl.kernel(
    _item_body,
    out_type=jax.ShapeDtypeStruct((NC * F,), jnp.float32),
    mesh=_SC_MESH,
    compiler_params=_SC_PARAMS,
    scratch_types=[
        pltpu.VMEM((IPW,), jnp.int32),
        pltpu.VMEM((F,), jnp.float32),
        pltpu.VMEM((NS * F,), jnp.float32),
        pltpu.VMEM_SHARED((NS * F,), jnp.float32),
        pltpu.SemaphoreType.DMA,
        pltpu.SemaphoreType.DMA,
        pltpu.VMEM((F, NV - TAIL), jnp.float32),
    ] + [pltpu.VMEM((F, CH), jnp.float32) for _ in range(NBUF)],
)


SCN = 262144
TCN = NV - SCN
BN = 131072
TCB0 = SCN // BN
NB_TC = (TCN + BN - 1) // BN
CPW = SCN // NW
CHK = 1024
NCHK = CPW // CHK
RING = 3


def _sweep_tc_body(s2_ref, ut_ref, y_ref):
  s = s2_ref[pl.ds(0, F)] + s2_ref[pl.ds(F, F)]
  y_ref[...] = jnp.dot(s.reshape(1, F), ut_ref[...],
                       preferred_element_type=jnp.float32).reshape(BN)


_sweep_tc_kernel = pl.pallas_call(
    _sweep_tc_body,
    out_shape=jax.ShapeDtypeStruct((TCN,), jnp.float32),
    grid=(NB_TC,),
    in_specs=[
        pl.BlockSpec((NC * F,), lambda j: (0,)),
        pl.BlockSpec((F, BN), lambda j: (0, j + TCB0)),
    ],
    out_specs=pl.BlockSpec((BN,), lambda j: (j,)),
)


def _sweep_sc_body(s2, uT, ysc, s_v, outv, sem, *rings):
  cid = lax.axis_index("c")
  sid = lax.axis_index("s")
  wid = sid * NC + cid
  base = wid * CPW
  zero = jnp.zeros((L,), jnp.float32)

  pltpu.sync_copy(s2, s_v)
  sa = s_v[pl.ds(0, L)] + s_v[pl.ds(F, L)]
  sb = s_v[pl.ds(L, L)] + s_v[pl.ds(F + L, L)]
  s_sc = [sa[l] for l in range(L)] + [sb[l] for l in range(L)]

  def fire(c):
    col = pl.multiple_of(base + c * CHK, CHK)
    return pltpu.async_copy(uT.at[:, pl.ds(col, CHK)], rings[c % RING], sem)

  copies = [fire(c) for c in range(RING)]
  for c in range(NCHK):
    copies[c % RING].wait()
    buf = rings[c % RING]

    @pl.loop(0, CHK // L, unroll=2)
    def _grp(b):
      acc = zero
      for f in range(F):
        acc = acc + buf[f, pl.ds(b * L, L)] * s_sc[f]
      outv[pl.ds(b * L, L)] = acc

    pltpu.sync_copy(outv, ysc.at[pl.ds(base + c * CHK, CHK)])
    if c + RING < NCHK:
      copies[c % RING] = fire(c + RING)


_sweep_sc_kernel = pl.kernel(
    _sweep_sc_body,
    out_type=jax.ShapeDtypeStruct((SCN,), jnp.float32),
    mesh=_SC_MESH,
    compiler_params=_SC_PARAMS,
    scratch_types=[
        pltpu.VMEM((NC * F,), jnp.float32),
        pltpu.VMEM((CHK,), jnp.float32),
        pltpu.SemaphoreType.DMA,
    ] + [pltpu.VMEM((F, CHK), jnp.float32) for _ in range(RING)],
)


def _pick_body(ysc, ytc, uidx, out, idx_v, idc_v, idt_v, yv_sc, yv_tc, outv,
               sem):
  cid = lax.axis_index("c")
  sid = lax.axis_index("s")
  wid = sid * NC + cid
  base = wid * UPW
  copies = []
  for t in range(N_UCH):
    pltpu.sync_copy(uidx.at[pl.ds(base + t * CH, CH)], idx_v.at[t])
    for k in range(CH // L):
      raw = idx_v[t, pl.ds(k * L, L)]
      in_sc = raw < SCN
      idc_v[t, pl.ds(k * L, L)] = jnp.where(in_sc, raw, raw & (SCN - 1))
      idt_v[t, pl.ds(k * L, L)] = jnp.where(in_sc, raw & (BN - 1), raw - SCN)
    copies.append(
        pltpu.async_copy(ysc.at[idc_v.at[t]], yv_sc.at[pl.ds(t * CH, CH)],
                         sem))
    copies.append(
        pltpu.async_copy(ytc.at[idt_v.at[t]], yv_tc.at[pl.ds(t * CH, CH)],
                         sem))
  for c in copies:
    c.wait()

  @pl.loop(0, UPW // L)
  def _sel(b):
    raw = idx_v[b // (CH // L), pl.ds((b % (CH // L)) * L, L)]
    vs = yv_sc[pl.ds(b * L, L)]
    vt = yv_tc[pl.ds(b * L, L)]
    outv[pl.ds(b * L, L)] = jnp.where(raw < SCN, vs, vt)

  pltpu.sync_copy(outv, out.at[pl.ds(base, UPW)])


_pick_kernel = pl.kernel(
    _pick_body,
    out_type=jax.ShapeDtypeStruct((B_USER,), jnp.float32),
    mesh=_SC_MESH,
    compiler_params=_SC_PARAMS,
    scratch_types=[
        pltpu.VMEM((N_UCH, CH), jnp.int32),
        pltpu.VMEM((N_UCH, CH), jnp.int32),
        pltpu.VMEM((N_UCH, CH), jnp.int32),
        pltpu.VMEM((UPW,), jnp.float32),
        pltpu.VMEM((UPW,), jnp.float32),
        pltpu.VMEM((UPW,), jnp.float32),
        pltpu.SemaphoreType.DMA,
    ],
)


def kernel(user_factors, item_factors, user_indices, item_indices):
  uT = user_factors.T
  itT = item_factors.T
  s2 = _item_kernel(itT, item_indices.astype(jnp.int32))
  y_sc = _sweep_sc_kernel(s2, uT)
  y_tc = _sweep_tc_kernel(s2, uT)
  return _pick_kernel(y_sc, y_tc, user_indices.astype(jnp.int32))


# speedup vs baseline: 1.0300x; 1.0300x over previous
import jax
import jax.numpy as jnp
from jax import lax
from jax.experimental import pallas as pl
from jax.experimental.pallas import tpu as pltpu
from jax.experimental.pallas import tpu_sc as plsc

F = 32
B_USER = 16384
B_ITEM = 4096
NV = 1000000
NC = 2
NS = 16
L = 16
NW = NC * NS
IPW = B_ITEM // NW
UPW = B_USER // NW
CH = 128
N_UCH = UPW // CH
NBUF = 16

_SC_PARAMS = pltpu.CompilerParams(
    needs_layout_passes=False, use_tc_tiling_on_sc=True)
_SC_MESH = plsc.VectorSubcoreMesh(core_axis_name="c", subcore_axis_name="s")


MAXC = (NV - CH) // CH * CH
TAIL = NV // CH * CH


def _item_body(itT, iidx, s2_out, idx_sm, part_v, ps_v, shared, sem, tsem,
               tail_v, *blks):
  cid = lax.axis_index("c")
  sid = lax.axis_index("s")
  wid = sid * NC + cid
  zero = jnp.zeros((L,), jnp.float32)
  lane = lax.iota(jnp.int32, L)

  pltpu.sync_copy(iidx.at[pl.ds(wid * IPW, IPW)], idx_sm)
  pltpu.async_copy(itT.at[:, pl.ds(TAIL, NV - TAIL)], tail_v, tsem).wait()

  js = []
  for b in range(IPW // L):
    jv = idx_sm[pl.ds(b * L, L)]
    js.extend(jv[l] for l in range(L))

  def aligned_col(j):
    return pl.multiple_of(
        jnp.minimum(j & ~jnp.int32(CH - 1), jnp.int32(MAXC)), CH)

  def fire(k):
    col = aligned_col(js[k])
    return pltpu.async_copy(itT.at[:, pl.ds(col, CH)], blks[k % NBUF], sem)

  copies = [fire(k) for k in range(NBUF)]
  a0, a1 = zero, zero
  for k in range(IPW):
    copies[k % NBUF].wait()
    j = js[k]
    col = aligned_col(j)
    is_tail = j >= TAIL
    sub = jnp.full((L,), jnp.minimum(j - col, CH - 1), jnp.int32)
    tsub = jnp.full((L,), jnp.clip(j - TAIL, 0, NV - TAIL - 1), jnp.int32)
    m0 = plsc.load_gather(blks[k % NBUF], [lane, sub])
    m1 = plsc.load_gather(blks[k % NBUF], [lane + L, sub])
    t0 = plsc.load_gather(tail_v, [lane, tsub])
    t1 = plsc.load_gather(tail_v, [lane + L, tsub])
    a0 = a0 + jnp.where(is_tail, t0, m0)
    a1 = a1 + jnp.where(is_tail, t1, m1)
    if k + NBUF < IPW:
      copies[k % NBUF] = fire(k + NBUF)

  part_v[pl.ds(0, L)] = a0
  part_v[pl.ds(L, L)] = a1
  pltpu.sync_copy(part_v, shared.at[pl.ds(sid * F, F)])
  plsc.subcore_barrier()
  pltpu.sync_copy(shared, ps_v)

  @pl.loop(0, NS, init_carry=(zero, zero), unroll=True)
  def _part_acc(i, carry):
    b0, b1 = carry
    return (b0 + ps_v[pl.ds(i * F, L)], b1 + ps_v[pl.ds(i * F + L, L)])
  s0, s1 = _part_acc

  @pl.when(sid == 0)
  def _():
    part_v[pl.ds(0, L)] = s0
    part_v[pl.ds(L, L)] = s1
    pltpu.sync_copy(part_v, s2_out.at[pl.ds(cid * F, F)])


_item_kernel = pl.kernel(
    _item_body,
    out_type=jax.ShapeDtypeStruct((NC * F,), jnp.float32),
    mesh=_SC_MESH,
    compiler_params=_SC_PARAMS,
    scratch_types=[
        pltpu.VMEM((IPW,), jnp.int32),
        pltpu.VMEM((F,), jnp.float32),
        pltpu.VMEM((NS * F,), jnp.float32),
        pltpu.VMEM_SHARED((NS * F,), jnp.float32),
        pltpu.SemaphoreType.DMA,
        pltpu.SemaphoreType.DMA,
        pltpu.VMEM((F, NV - TAIL), jnp.float32),
    ] + [pltpu.VMEM((F, CH), jnp.float32) for _ in range(NBUF)],
)


BN = 131072
NB = (NV + BN - 1) // BN


def _sweep_body(s2_ref, ut_ref, y_ref):
  s = s2_ref[pl.ds(0, F)] + s2_ref[pl.ds(F, F)]
  y_ref[...] = jnp.dot(s.reshape(1, F), ut_ref[...],
                       preferred_element_type=jnp.float32).reshape(BN)


_sweep_kernel = pl.pallas_call(
    _sweep_body,
    out_shape=jax.ShapeDtypeStruct((NV,), jnp.float32),
    grid=(NB,),
    in_specs=[
        pl.BlockSpec((NC * F,), lambda j: (0,)),
        pl.BlockSpec((F, BN), lambda j: (0, j)),
    ],
    out_specs=pl.BlockSpec((BN,), lambda j: (j,)),
)


def _pick_body(y, uidx, out, idx_v, yv, sem):
  cid = lax.axis_index("c")
  sid = lax.axis_index("s")
  wid = sid * NC + cid
  base = wid * UPW
  copies = []
  for t in range(N_UCH):
    pltpu.sync_copy(uidx.at[pl.ds(base + t * CH, CH)], idx_v.at[t])
    copies.append(
        pltpu.async_copy(y.at[idx_v.at[t]], yv.at[pl.ds(t * CH, CH)], sem))
  for c in copies:
    c.wait()
  pltpu.sync_copy(yv, out.at[pl.ds(base, UPW)])


_pick_kernel = pl.kernel(
    _pick_body,
    out_type=jax.ShapeDtypeStruct((B_USER,), jnp.float32),
    mesh=_SC_MESH,
    compiler_params=_SC_PARAMS,
    scratch_types=[
        pltpu.VMEM((N_UCH, CH), jnp.int32),
        pltpu.VMEM((UPW,), jnp.float32),
        pltpu.SemaphoreType.DMA,
    ],
)


def kernel(user_factors, item_factors, user_indices, item_indices):
  uT = user_factors.T
  itT = item_factors.T
  s2 = _item_kernel(itT, item_indices.astype(jnp.int32))
  y = _sweep_kernel(s2, uT)
  return _pick_kernel(y, user_indices.astype(jnp.int32))


# speedup vs baseline: 1.0514x; 1.0208x over previous
import jax
import jax.numpy as jnp
from jax import lax
from jax.experimental import pallas as pl
from jax.experimental.pallas import tpu as pltpu
from jax.experimental.pallas import tpu_sc as plsc

F = 32
B_USER = 16384
B_ITEM = 4096
NV = 1000000
NC = 2
NS = 16
L = 16
NW = NC * NS
IPW = B_ITEM // NW
UPW = B_USER // NW
CH = 128
N_UCH = UPW // CH
NBUF = 16

_SC_PARAMS = pltpu.CompilerParams(
    needs_layout_passes=False, use_tc_tiling_on_sc=True)
_SC_MESH = plsc.VectorSubcoreMesh(core_axis_name="c", subcore_axis_name="s")


MAXC = (NV - CH) // CH * CH
TAIL = NV // CH * CH


def _item_body(itT, iidx, s2_out, idx_sm, part_v, ps_v, shared, sem, tsem,
               tail_v, *blks):
  cid = lax.axis_index("c")
  sid = lax.axis_index("s")
  wid = sid * NC + cid
  zero = jnp.zeros((L,), jnp.float32)
  lane = lax.iota(jnp.int32, L)

  pltpu.sync_copy(iidx.at[pl.ds(wid * IPW, IPW)], idx_sm)
  pltpu.async_copy(itT.at[:, pl.ds(TAIL, NV - TAIL)], tail_v, tsem).wait()

  js = []
  for b in range(IPW // L):
    jv = idx_sm[pl.ds(b * L, L)]
    js.extend(jv[l] for l in range(L))

  def aligned_col(j):
    return pl.multiple_of(
        jnp.minimum(j & ~jnp.int32(CH - 1), jnp.int32(MAXC)), CH)

  def fire(k):
    col = aligned_col(js[k])
    return pltpu.async_copy(itT.at[:, pl.ds(col, CH)], blks[k % NBUF], sem)

  copies = [fire(k) for k in range(NBUF)]
  a0, a1 = zero, zero
  for k in range(IPW):
    copies[k % NBUF].wait()
    j = js[k]
    col = aligned_col(j)
    is_tail = j >= TAIL
    sub = jnp.full((L,), jnp.minimum(j - col, CH - 1), jnp.int32)
    tsub = jnp.full((L,), jnp.clip(j - TAIL, 0, NV - TAIL - 1), jnp.int32)
    m0 = plsc.load_gather(blks[k % NBUF], [lane, sub])
    m1 = plsc.load_gather(blks[k % NBUF], [lane + L, sub])
    t0 = plsc.load_gather(tail_v, [lane, tsub])
    t1 = plsc.load_gather(tail_v, [lane + L, tsub])
    a0 = a0 + jnp.where(is_tail, t0, m0)
    a1 = a1 + jnp.where(is_tail, t1, m1)
    if k + NBUF < IPW:
      copies[k % NBUF] = fire(k + NBUF)

  part_v[pl.ds(0, L)] = a0
  part_v[pl.ds(L, L)] = a1
  pltpu.sync_copy(part_v, shared.at[pl.ds(sid * F, F)])
  plsc.subcore_barrier()
  pltpu.sync_copy(shared, ps_v)

  @pl.loop(0, NS, init_carry=(zero, zero), unroll=True)
  def _part_acc(i, carry):
    b0, b1 = carry
    return (b0 + ps_v[pl.ds(i * F, L)], b1 + ps_v[pl.ds(i * F + L, L)])
  s0, s1 = _part_acc

  @pl.when(sid == 0)
  def _():
    part_v[pl.ds(0, L)] = s0
    part_v[pl.ds(L, L)] = s1
    pltpu.sync_copy(part_v, s2_out.at[pl.ds(cid * F, F)])


_item_kernel = pl.kernel(
    _item_body,
    out_type=jax.ShapeDtypeStruct((NC * F,), jnp.float32),
    mesh=_SC_MESH,
    compiler_params=_SC_PARAMS,
    scratch_types=[
        pltpu.VMEM((IPW,), jnp.int32),
        pltpu.VMEM((F,), jnp.float32),
        pltpu.VMEM((NS * F,), jnp.float32),
        pltpu.VMEM_SHARED((NS * F,), jnp.float32),
        pltpu.SemaphoreType.DMA,
        pltpu.SemaphoreType.DMA,
        pltpu.VMEM((F, NV - TAIL), jnp.float32),
    ] + [pltpu.VMEM((F, CH), jnp.float32) for _ in range(NBUF)],
)


BN = 65536
NB = (NV + BN - 1) // BN


def _sweep_body(s2_ref, ut_ref, y_ref):
  s = s2_ref[pl.ds(0, F)] + s2_ref[pl.ds(F, F)]
  y_ref[...] = jnp.dot(s.reshape(1, F), ut_ref[...],
                       preferred_element_type=jnp.float32).reshape(BN)


_sweep_kernel = pl.pallas_call(
    _sweep_body,
    out_shape=jax.ShapeDtypeStruct((NV,), jnp.float32),
    grid=(NB,),
    in_specs=[
        pl.BlockSpec((NC * F,), lambda j: (0,)),
        pl.BlockSpec((F, BN), lambda j: (0, j)),
    ],
    out_specs=pl.BlockSpec((BN,), lambda j: (j,)),
)


def _pick_body(y, uidx, out, idx_v, yv, sem):
  cid = lax.axis_index("c")
  sid = lax.axis_index("s")
  wid = sid * NC + cid
  base = wid * UPW
  copies = []
  for t in range(N_UCH):
    pltpu.sync_copy(uidx.at[pl.ds(base + t * CH, CH)], idx_v.at[t])
    copies.append(
        pltpu.async_copy(y.at[idx_v.at[t]], yv.at[pl.ds(t * CH, CH)], sem))
  for c in copies:
    c.wait()
  pltpu.sync_copy(yv, out.at[pl.ds(base, UPW)])


_pick_kernel = pl.kernel(
    _pick_body,
    out_type=jax.ShapeDtypeStruct((B_USER,), jnp.float32),
    mesh=_SC_MESH,
    compiler_params=_SC_PARAMS,
    scratch_types=[
        pltpu.VMEM((N_UCH, CH), jnp.int32),
        pltpu.VMEM((UPW,), jnp.float32),
        pltpu.SemaphoreType.DMA,
    ],
)


def kernel(user_factors, item_factors, user_indices, item_indices):
  uT = user_factors.T
  itT = item_factors.T
  s2 = _item_kernel(itT, item_indices.astype(jnp.int32))
  y = _sweep_kernel(s2, uT)
  return _pick_kernel(y, user_indices.astype(jnp.int32))
